# baseline (speedup 1.0000x reference)
import functools

import jax
import jax.numpy as jnp
from jax import lax
from jax.experimental import pallas as pl
from jax.experimental.pallas import tpu as pltpu
from jax.experimental.pallas import tpu_sc as plsc

TOPK = 5
TPAD = 8
KB = 2048
BIG = 1e30
IBIG = 2**31 - 1


def _stage_a_body(kreal, k_ref, qt_ref, cd_ref, ci_ref):
    b = pl.program_id(0)
    qt = qt_ref[...]
    kb = k_ref[...]
    qn = qt.shape[1]
    mm = lax.dot_general(kb, qt, (((1,), (0,)), ((), ())),
                         preferred_element_type=jnp.float32)
    qsq = jnp.sum(qt * qt, axis=0, keepdims=True)
    ksq = jnp.sum(kb * kb, axis=1, keepdims=True)
    row = lax.broadcasted_iota(jnp.int32, (KB, 1), 0) + b * KB
    pen = jnp.where(row >= kreal, jnp.float32(BIG), jnp.float32(0.0))
    s = (qsq + (ksq + pen)) - 2.0 * mm
    srow = lax.broadcasted_iota(jnp.int32, (TPAD, qn), 0)
    res_d = jnp.full((TPAD, qn), jnp.float32(BIG))
    res_i = jnp.full((TPAD, qn), IBIG, jnp.int32)
    for t in range(TOPK):
        m = jnp.min(s, axis=0, keepdims=True)
        eq = s == m
        am = jnp.min(jnp.where(eq, row, IBIG), axis=0, keepdims=True)
        res_d = jnp.where(srow == t, m, res_d)
        res_i = jnp.where(srow == t, am, res_i)
        if t < TOPK - 1:
            s = jnp.where(row == am, jnp.float32(BIG), s)
    cd_ref[...] = res_d.reshape(1, TPAD, qn)
    ci_ref[...] = res_i.reshape(1, TPAD, qn)


def _stage_b_body(cd_ref, ci_ref, od_ref, oi_ref):
    nb, tpad, qn = cd_ref.shape
    cd = cd_ref[...].reshape(nb * tpad, qn)
    ci = ci_ref[...].reshape(nb * tpad, qn)
    for t in range(TOPK):
        m = jnp.min(cd, axis=0, keepdims=True)
        eq = cd == m
        am = jnp.min(jnp.where(eq, ci, IBIG), axis=0, keepdims=True)
        od_ref[t:t + 1, :] = m
        oi_ref[t:t + 1, :] = am
        if t < TOPK - 1:
            cd = jnp.where(eq & (ci == am), jnp.float32(BIG), cd)


def _topk_tc(queries, keys):
    qn, d = queries.shape
    kreal = keys.shape[0]
    nb = (kreal + KB - 1) // KB
    kpad = nb * KB
    if kpad != kreal:
        keys = jnp.pad(keys, ((0, kpad - kreal), (0, 0)))

    cand_d, cand_i = pl.pallas_call(
        functools.partial(_stage_a_body, kreal),
        grid=(nb,),
        in_specs=[
            pl.BlockSpec((KB, d), lambda b: (b, 0)),
            pl.BlockSpec((d, qn), lambda b: (0, 0)),
        ],
        out_specs=[
            pl.BlockSpec((1, TPAD, qn), lambda b: (b, 0, 0)),
            pl.BlockSpec((1, TPAD, qn), lambda b: (b, 0, 0)),
        ],
        out_shape=[
            jax.ShapeDtypeStruct((nb, TPAD, qn), jnp.float32),
            jax.ShapeDtypeStruct((nb, TPAD, qn), jnp.int32),
        ],
        compiler_params=pltpu.CompilerParams(
            dimension_semantics=("arbitrary",)),
    )(keys, queries.T)

    od, oi = pl.pallas_call(
        _stage_b_body,
        out_shape=[
            jax.ShapeDtypeStruct((TOPK, qn), jnp.float32),
            jax.ShapeDtypeStruct((TOPK, qn), jnp.int32),
        ],
    )(cand_d, cand_i)
    return od.T, oi.T


def _gather_mean_sc(values, idx_flat, qn):
    info = plsc.get_sparse_core_info()
    nc, ns = info.num_cores, info.num_subcores
    nw = nc * ns
    d = values.shape[1]
    qpw = qn // nw
    rpw = qpw * TOPK
    mesh = plsc.VectorSubcoreMesh(core_axis_name="c", subcore_axis_name="s")

    @functools.partial(
        pl.kernel, mesh=mesh,
        out_type=jax.ShapeDtypeStruct((qn, d), jnp.float32),
        scratch_types=[
            pltpu.VMEM((rpw,), jnp.int32),
            pltpu.VMEM((rpw, d), jnp.float32),
            pltpu.VMEM((qpw, d), jnp.float32),
            pltpu.SemaphoreType.DMA,
        ],
    )
    def gather_kernel(values_hbm, idx_hbm, out_hbm, idx_v, rows_v, acc_v, sem):
        wid = lax.axis_index("s") * nc + lax.axis_index("c")
        pltpu.sync_copy(idx_hbm.at[pl.ds(wid * rpw, rpw)], idx_v)
        pltpu.async_copy(values_hbm.at[idx_v], rows_v, sem).wait()

        def body(qi, _):
            for c in range(d // 16):
                sl = pl.ds(c * 16, 16)
                acc = rows_v[qi * TOPK, sl]
                for j in range(1, TOPK):
                    acc = acc + rows_v[qi * TOPK + j, sl]
                acc_v[qi, sl] = acc * jnp.float32(1.0 / TOPK)
            return 0

        lax.fori_loop(0, qpw, body, 0)
        pltpu.sync_copy(acc_v, out_hbm.at[pl.ds(wid * qpw, qpw)])

    return gather_kernel(values, idx_flat)


def kernel(queries, keys, values, k):
    topk_d, topk_i = _topk_tc(queries, keys)
    retrieved = _gather_mean_sc(values, topk_i.reshape(-1), queries.shape[0])
    idx = topk_i + (k - TOPK)
    return retrieved, topk_d, idx


# speedup vs baseline: 1.7059x; 1.7059x over previous
import functools

import jax
import jax.numpy as jnp
from jax import lax
from jax.experimental import pallas as pl
from jax.experimental.pallas import tpu as pltpu
from jax.experimental.pallas import tpu_sc as plsc

TOPK = 5
TPAD = 8
KB = 2000
BIG = 1e30
IBIG = 2**31 - 1


def _stage_a_body(kreal, k_ref, qt_ref, cd_ref, ci_ref):
    b = pl.program_id(0)
    qt = qt_ref[...]
    kb = k_ref[...]
    qn = qt.shape[1]
    mm = lax.dot_general(kb, qt, (((1,), (0,)), ((), ())),
                         preferred_element_type=jnp.float32)
    qsq = jnp.sum(qt * qt, axis=0, keepdims=True)
    ksq = jnp.sum(kb * kb, axis=1, keepdims=True)
    rowc = lax.broadcasted_iota(jnp.int32, (KB, 1), 0) + b * KB
    if kreal % KB != 0:
        ksq = ksq + jnp.where(rowc >= kreal,
                              jnp.float32(BIG), jnp.float32(0.0))
    s0 = (qsq + ksq) - 2.0 * mm
    li = lax.broadcasted_iota(jnp.int32, (1, KB), 1)
    lhs = jnp.concatenate(
        [(li // 64).astype(jnp.float32),
         (li % 64).astype(jnp.float32),
         jnp.ones((1, KB), jnp.float32)], axis=0)
    srow = lax.broadcasted_iota(jnp.int32, (TPAD, qn), 0)
    res_d = jnp.full((TPAD, qn), jnp.float32(BIG))
    res_i = jnp.full((TPAD, qn), IBIG, jnp.int32)
    cmax = jnp.zeros((1, qn), jnp.float32)
    s = s0
    for t in range(TOPK):
        m = jnp.min(s, axis=0, keepdims=True)
        eq = s == m
        eqf = jnp.where(eq, jnp.float32(1.0), jnp.float32(0.0))
        rc = lax.dot_general(lhs, eqf, (((1,), (0,)), ((), ())),
                             preferred_element_type=jnp.float32)
        am = (rc[0:1, :] * 64.0 + rc[1:2, :]).astype(jnp.int32) + b * KB
        cmax = jnp.maximum(cmax, rc[2:3, :])
        res_d = jnp.where(srow == t, m, res_d)
        res_i = jnp.where(srow == t, am, res_i)
        if t < TOPK - 1:
            s = jnp.where(eq, jnp.float32(BIG), s)
    cd_ref[...] = res_d.reshape(1, TPAD, qn)
    ci_ref[...] = res_i.reshape(1, TPAD, qn)

    @pl.when(jnp.max(cmax) > 1.5)
    def _slow_path():
        sd = s0
        rd = jnp.full((TPAD, qn), jnp.float32(BIG))
        ri = jnp.full((TPAD, qn), IBIG, jnp.int32)
        for t in range(TOPK):
            m = jnp.min(sd, axis=0, keepdims=True)
            eq = sd == m
            am = jnp.min(jnp.where(eq, rowc, IBIG), axis=0, keepdims=True)
            rd = jnp.where(srow == t, m, rd)
            ri = jnp.where(srow == t, am, ri)
            if t < TOPK - 1:
                sd = jnp.where(rowc == am, jnp.float32(BIG), sd)
        cd_ref[...] = rd.reshape(1, TPAD, qn)
        ci_ref[...] = ri.reshape(1, TPAD, qn)


def _stage_b_body(cd_ref, ci_ref, od_ref, oi_ref):
    nb, tpad, qn = cd_ref.shape
    cd = cd_ref[...].reshape(nb * tpad, qn)
    ci = ci_ref[...].reshape(nb * tpad, qn)
    for t in range(TOPK):
        m = jnp.min(cd, axis=0, keepdims=True)
        eq = cd == m
        am = jnp.min(jnp.where(eq, ci, IBIG), axis=0, keepdims=True)
        od_ref[t:t + 1, :] = m
        oi_ref[t:t + 1, :] = am
        if t < TOPK - 1:
            cd = jnp.where(eq & (ci == am), jnp.float32(BIG), cd)


def _topk_tc(queries, keys):
    qn, d = queries.shape
    kreal = keys.shape[0]
    nb = (kreal + KB - 1) // KB
    kpad = nb * KB
    if kpad != kreal:
        keys = jnp.pad(keys, ((0, kpad - kreal), (0, 0)))

    cand_d, cand_i = pl.pallas_call(
        functools.partial(_stage_a_body, kreal),
        grid=(nb,),
        in_specs=[
            pl.BlockSpec((KB, d), lambda b: (b, 0)),
            pl.BlockSpec((d, qn), lambda b: (0, 0)),
        ],
        out_specs=[
            pl.BlockSpec((1, TPAD, qn), lambda b: (b, 0, 0)),
            pl.BlockSpec((1, TPAD, qn), lambda b: (b, 0, 0)),
        ],
        out_shape=[
            jax.ShapeDtypeStruct((nb, TPAD, qn), jnp.float32),
            jax.ShapeDtypeStruct((nb, TPAD, qn), jnp.int32),
        ],
        compiler_params=pltpu.CompilerParams(
            dimension_semantics=("arbitrary",)),
    )(keys, queries.T)

    od, oi = pl.pallas_call(
        _stage_b_body,
        out_shape=[
            jax.ShapeDtypeStruct((TOPK, qn), jnp.float32),
            jax.ShapeDtypeStruct((TOPK, qn), jnp.int32),
        ],
    )(cand_d, cand_i)
    return od.T, oi.T


def _gather_mean_sc(values, idx_flat, qn):
    info = plsc.get_sparse_core_info()
    nc, ns = info.num_cores, info.num_subcores
    nw = nc * ns
    d = values.shape[1]
    qpw = qn // nw
    rpw = qpw * TOPK
    mesh = plsc.VectorSubcoreMesh(core_axis_name="c", subcore_axis_name="s")

    @functools.partial(
        pl.kernel, mesh=mesh,
        out_type=jax.ShapeDtypeStruct((qn, d), jnp.float32),
        scratch_types=[
            pltpu.VMEM((rpw,), jnp.int32),
            pltpu.VMEM((rpw, d), jnp.float32),
            pltpu.VMEM((qpw, d), jnp.float32),
            pltpu.SemaphoreType.DMA,
        ],
    )
    def gather_kernel(values_hbm, idx_hbm, out_hbm, idx_v, rows_v, acc_v, sem):
        wid = lax.axis_index("s") * nc + lax.axis_index("c")
        pltpu.sync_copy(idx_hbm.at[pl.ds(wid * rpw, rpw)], idx_v)
        pltpu.async_copy(values_hbm.at[idx_v], rows_v, sem).wait()

        def body(qi, _):
            for c in range(d // 16):
                sl = pl.ds(c * 16, 16)
                acc = rows_v[qi * TOPK, sl]
                for j in range(1, TOPK):
                    acc = acc + rows_v[qi * TOPK + j, sl]
                acc_v[qi, sl] = acc * jnp.float32(1.0 / TOPK)
            return 0

        lax.fori_loop(0, qpw, body, 0)
        pltpu.sync_copy(acc_v, out_hbm.at[pl.ds(wid * qpw, qpw)])

    return gather_kernel(values, idx_flat)


def kernel(queries, keys, values, k):
    topk_d, topk_i = _topk_tc(queries, keys)
    retrieved = _gather_mean_sc(values, topk_i.reshape(-1), queries.shape[0])
    idx = topk_i + (k - TOPK)
    return retrieved, topk_d, idx
